# baseline (speedup 1.0000x reference)
import functools

import jax
import jax.numpy as jnp
from jax import lax
from jax.experimental import pallas as pl
from jax.experimental.pallas import tpu as pltpu
from jax.experimental.pallas import tpu_sc as plsc

_Q = 1024
_D = 16
_DA = 24
_KB = 1024
_PAD = 1e18
_TAG_BITS = 7
_TAG_MASK = (1 << _TAG_BITS) - 1
_T = 1.00000005e-4
_NC = 2
_NS = 16


def _aug_body(kt_ref, kaug_ref):
    kt = kt_ref[...]
    ksq = jnp.sum(kt * kt, axis=0, keepdims=True)
    kaug_ref[...] = jnp.concatenate(
        [kt * (-2.0), ksq, jnp.zeros((_DA - _D - 1, kt.shape[1]), jnp.float32)],
        axis=0)


def _augment_keys(keys_t, nsteps):
    return pl.pallas_call(
        _aug_body,
        grid=(nsteps,),
        in_specs=[pl.BlockSpec((_D, _KB), lambda j: (0, j))],
        out_specs=pl.BlockSpec((_DA, _KB), lambda j: (0, j)),
        out_shape=jax.ShapeDtypeStruct((_DA, nsteps * _KB), jnp.float32),
    )(keys_t)


def _tc_body(qaug_ref, kaug_ref, idx_ref, racc, *, nsteps, kb):
    j = pl.program_id(0)
    s = jnp.dot(qaug_ref[...], kaug_ref[...],
                preferred_element_type=jnp.float32)
    bits = lax.bitcast_convert_type(s, jnp.int32)
    tagged = lax.bitcast_convert_type((bits & jnp.int32(~_TAG_MASK)) | j,
                                      jnp.float32)

    @pl.when(j == 0)
    def _():
        racc[...] = tagged

    @pl.when(j > 0)
    def _():
        racc[...] = jnp.minimum(racc[...], tagged)

    @pl.when(j == nsteps - 1)
    def _():
        r = racc[...]
        rmin = jnp.min(r, axis=1, keepdims=True)
        rbits = lax.bitcast_convert_type(r, jnp.int32)
        lane = lax.broadcasted_iota(jnp.int32, r.shape, 1)
        gidx = (rbits & _TAG_MASK) * kb + lane
        cand = jnp.where(r == rmin, gidx, jnp.int32(2**31 - 1))
        idx_ref[...] = jnp.min(cand, axis=1, keepdims=True)


def _tc_argmin(queries_aug, keys_aug, nsteps):
    return pl.pallas_call(
        functools.partial(_tc_body, nsteps=nsteps, kb=_KB),
        grid=(nsteps,),
        in_specs=[
            pl.BlockSpec((_Q, _DA), lambda j: (0, 0)),
            pl.BlockSpec((_DA, _KB), lambda j: (0, j)),
        ],
        out_specs=pl.BlockSpec((_Q, 1), lambda j: (0, 0)),
        out_shape=jax.ShapeDtypeStruct((_Q, 1), jnp.int32),
        scratch_shapes=[
            pltpu.VMEM((_Q, _KB), jnp.float32),
        ],
    )(queries_aug, keys_aug)


_CH = _Q // (_NC * _NS)


@functools.cache
def _make_sc_verify_gather():
    @functools.partial(
        pl.kernel,
        out_type=jax.ShapeDtypeStruct((_Q,), jnp.float32),
        mesh=plsc.VectorSubcoreMesh(core_axis_name="c", subcore_axis_name="s",
                                    num_cores=_NC, num_subcores=_NS),
        scratch_types=[
            pltpu.VMEM((_CH,), jnp.int32),
            pltpu.VMEM((_CH * _D,), jnp.int32),
            pltpu.VMEM((_CH,), jnp.float32),
            pltpu.VMEM((_CH * _D,), jnp.float32),
            pltpu.VMEM((_CH * _D,), jnp.float32),
            pltpu.VMEM((_CH,), jnp.float32),
            pltpu.SemaphoreType.DMA,
        ],
    )
    def _sc_verify_gather(idx_hbm, queries_t_hbm, keys_flat_hbm, values_hbm,
                          out_hbm, idx_v, gidx_v, val_v, qt_v, kgat_v, out_v,
                          sem):
        wid = lax.axis_index("s") * _NC + lax.axis_index("c")
        base = wid * _CH
        pltpu.sync_copy(idx_hbm.at[pl.ds(base, _CH)], idx_v)
        pltpu.async_copy(values_hbm.at[idx_v], val_v, sem).wait()
        for c in range(_D):
            pltpu.sync_copy(queries_t_hbm.at[pl.ds(c * _Q + base, _CH)],
                            qt_v.at[pl.ds(c * _CH, _CH)])
        half = [idx_v[pl.ds(0, 16)] * _D, idx_v[pl.ds(16, 16)] * _D]
        for c in range(_D):
            for h in range(_CH // 16):
                gidx_v[pl.ds(c * _CH + h * 16, 16)] = half[h] + c
        for b in range(_CH * _D // 128):
            pltpu.async_copy(
                keys_flat_hbm.at[gidx_v.at[pl.ds(b * 128, 128)]],
                kgat_v.at[pl.ds(b * 128, 128)], sem).wait()
        for t in range(_CH // 16):
            acc = jnp.zeros((16,), jnp.float32)
            for c in range(_D):
                sl = pl.ds(c * _CH + t * 16, 16)
                dv = kgat_v[sl] - qt_v[sl]
                acc = acc + dv * dv
            osl = pl.ds(t * 16, 16)
            out_v[osl] = jnp.where(acc <= _T, val_v[osl], 0.0)
        pltpu.sync_copy(out_v, out_hbm.at[pl.ds(base, _CH)])

    return _sc_verify_gather


def kernel(queries, keys, values):
    k = keys.shape[0]
    nsteps = -(-k // _KB)
    kp = nsteps * _KB
    keys_t = jnp.pad(keys, ((0, kp - k), (0, 0)), constant_values=_PAD).T
    keys_aug = _augment_keys(keys_t, nsteps)
    queries_aug = jnp.pad(
        jnp.concatenate([queries, jnp.ones((_Q, 1), jnp.float32)], axis=1),
        ((0, 0), (0, _DA - _D - 1)))
    idx = _tc_argmin(queries_aug, keys_aug, nsteps)
    return _make_sc_verify_gather()(
        idx.reshape(_Q), queries.T.reshape(-1), keys.reshape(-1), values)
